# baseline (speedup 1.0000x reference)
import functools

import jax
import jax.numpy as jnp
from jax import lax
from jax.experimental import pallas as pl
from jax.experimental.pallas import tpu as pltpu
from jax.experimental.pallas import tpu_sc as plsc

D_MODEL = 1024
HIDDEN = 2048
N_EXP = 8
N_TOK = 4096
BT = 128
NB = N_TOK // BT
NW = NB + N_EXP - 1

RB = 2048
NRB = N_TOK // RB

SC_WORKERS = 32
ROWS_W = N_TOK // SC_WORKERS
SC_CHUNK = 64
N_CHUNK = ROWS_W // SC_CHUNK


def _router_body(x_ref, gwt_ref, top1_ref, rank_ref, counts_ref, carry):
    b = pl.program_id(0)

    @pl.when(b == 0)
    def _():
        carry[...] = jnp.zeros_like(carry)

    logits = jnp.dot(x_ref[...], gwt_ref[...],
                     preferred_element_type=jnp.float32)
    m = jnp.max(logits, axis=1, keepdims=True)
    col = lax.broadcasted_iota(jnp.int32, (RB, N_EXP), 1)
    top1 = jnp.min(jnp.where(logits == m, col, N_EXP), axis=1)
    oh = (col == top1[:, None]).astype(jnp.int32)
    c = oh
    s = 1
    while s < RB:
        z = jnp.zeros((s, N_EXP), jnp.int32)
        c = c + jnp.concatenate([z, c[:-s]], axis=0)
        s *= 2
    cum = c + carry[...]
    rank = jnp.sum(jnp.where(oh == 1, cum, 0), axis=1) - 1
    top1_ref[...] = top1[:, None]
    rank_ref[...] = rank[:, None]
    carry[...] = cum[-1:, :]
    counts_ref[...] = cum[-1:, :]


def _router(xt, gate_w):
    return pl.pallas_call(
        _router_body,
        grid=(NRB,),
        in_specs=[
            pl.BlockSpec((RB, D_MODEL), lambda i: (i, 0)),
            pl.BlockSpec((D_MODEL, N_EXP), lambda i: (0, 0)),
        ],
        out_specs=[
            pl.BlockSpec((RB, 1), lambda i: (i, 0)),
            pl.BlockSpec((RB, 1), lambda i: (i, 0)),
            pl.BlockSpec((1, N_EXP), lambda i: (0, 0)),
        ],
        out_shape=[
            jax.ShapeDtypeStruct((N_TOK, 1), jnp.int32),
            jax.ShapeDtypeStruct((N_TOK, 1), jnp.int32),
            jax.ShapeDtypeStruct((1, N_EXP), jnp.int32),
        ],
        scratch_shapes=[pltpu.VMEM((1, N_EXP), jnp.int32)],
    )(xt, gate_w.T)


def _work_plan(counts):
    ends = jnp.cumsum(counts)
    starts = ends - counts
    bpos = jnp.arange(NB, dtype=jnp.int32) * BT
    e_lo = jnp.sum((ends[None, :] <= bpos[:, None]).astype(jnp.int32), axis=1)
    e_hi = jnp.sum((ends[None, :] <= (bpos + BT - 1)[:, None]).astype(jnp.int32),
                   axis=1)
    per_block = e_hi - e_lo + 1
    cpb = jnp.cumsum(per_block)
    cpb_prev = cpb - per_block
    nval = cpb[NB - 1]
    w = jnp.arange(NW, dtype=jnp.int32)
    wc = jnp.minimum(w, nval - 1)
    bid = jnp.sum((cpb[None, :] <= wc[:, None]).astype(jnp.int32), axis=1)
    eid = e_lo[bid] + (wc - cpb_prev[bid])
    lo = jnp.maximum(starts[eid], bid * BT) - bid * BT
    hi = jnp.minimum(ends[eid], bid * BT + BT) - bid * BT
    pad = w >= nval
    lo = jnp.where(pad, 0, lo).astype(jnp.int32)
    hi = jnp.where(pad, 0, hi).astype(jnp.int32)
    first = jnp.concatenate([
        jnp.ones((1,), jnp.int32),
        (bid[1:] != bid[:-1]).astype(jnp.int32),
    ])
    return jnp.stack([bid.astype(jnp.int32), eid.astype(jnp.int32),
                      lo, hi, first])


def _sc_mesh():
    return plsc.VectorSubcoreMesh(core_axis_name="c", subcore_axis_name="s")


def _dispatch_body(x_hbm, idx_hbm, out_hbm, idx_v, rows_v, sem):
    wid = lax.axis_index("s") * 2 + lax.axis_index("c")
    base = wid * ROWS_W
    pltpu.sync_copy(idx_hbm.at[wid], idx_v)
    for ch in range(N_CHUNK):
        pltpu.sync_copy(x_hbm.at[pl.ds(base + ch * SC_CHUNK, SC_CHUNK)], rows_v)
        pltpu.async_copy(rows_v, out_hbm.at[idx_v.at[ch]], sem).wait()


def _sc_dispatch(xt, idx3):
    k = functools.partial(
        pl.kernel,
        out_type=jax.ShapeDtypeStruct((N_TOK, D_MODEL), jnp.float32),
        mesh=_sc_mesh(),
        scratch_types=[
            pltpu.VMEM((N_CHUNK, SC_CHUNK), jnp.int32),
            pltpu.VMEM((SC_CHUNK, D_MODEL), jnp.float32),
            pltpu.SemaphoreType.DMA,
        ],
    )(_dispatch_body)
    return k(xt, idx3)


def _combine_body(ys_hbm, idx_hbm, out_hbm, idx_v, rows_v, sem):
    wid = lax.axis_index("s") * 2 + lax.axis_index("c")
    base = wid * ROWS_W
    pltpu.sync_copy(idx_hbm.at[wid], idx_v)
    for ch in range(N_CHUNK):
        pltpu.async_copy(ys_hbm.at[idx_v.at[ch]], rows_v, sem).wait()
        pltpu.sync_copy(rows_v, out_hbm.at[pl.ds(base + ch * SC_CHUNK, SC_CHUNK)])


def _sc_combine(ys, idx3):
    k = functools.partial(
        pl.kernel,
        out_type=jax.ShapeDtypeStruct((N_TOK, D_MODEL), jnp.float32),
        mesh=_sc_mesh(),
        scratch_types=[
            pltpu.VMEM((N_CHUNK, SC_CHUNK), jnp.int32),
            pltpu.VMEM((SC_CHUNK, D_MODEL), jnp.float32),
            pltpu.SemaphoreType.DMA,
        ],
    )(_combine_body)
    return k(ys, idx3)


def _ffn_body(meta_ref, xs_ref, lng_ref, lnb_ref, w1_ref, b1_ref, w2_ref,
              b2_ref, out_ref):
    w = pl.program_id(0)
    lo = meta_ref[2, w]
    hi = meta_ref[3, w]
    first = meta_ref[4, w]

    @pl.when(first == 1)
    def _():
        out_ref[...] = jnp.zeros_like(out_ref)

    @pl.when(hi > lo)
    def _():
        xb = xs_ref[...]
        mu = jnp.mean(xb, axis=1, keepdims=True)
        var = jnp.mean(jnp.square(xb - mu), axis=1, keepdims=True)
        xhat = (xb - mu) * lax.rsqrt(var + 1e-5)
        ln = xhat * lng_ref[0, 0] + lnb_ref[0, 0]
        h = jnp.dot(ln, w1_ref[0], preferred_element_type=jnp.float32)
        h = h + b1_ref[0, 0]
        h = 0.5 * h * (1.0 + lax.erf(h * 0.7071067811865476))
        o = jnp.dot(h, w2_ref[0], preferred_element_type=jnp.float32)
        o = o + b2_ref[0, 0]
        ridx = lax.broadcasted_iota(jnp.int32, (BT, 1), 0)
        mask = (ridx >= lo) & (ridx < hi)
        out_ref[...] += jnp.where(mask, o, 0.0)


def _grouped_ffn(meta, xs, ln_g, ln_b, w1, b1, w2, b2):
    grid_spec = pltpu.PrefetchScalarGridSpec(
        num_scalar_prefetch=1,
        grid=(NW,),
        in_specs=[
            pl.BlockSpec((BT, D_MODEL), lambda w, m: (m[0, w], 0)),
            pl.BlockSpec((1, 1, D_MODEL), lambda w, m: (m[1, w], 0, 0)),
            pl.BlockSpec((1, 1, D_MODEL), lambda w, m: (m[1, w], 0, 0)),
            pl.BlockSpec((1, D_MODEL, HIDDEN), lambda w, m: (m[1, w], 0, 0)),
            pl.BlockSpec((1, 1, HIDDEN), lambda w, m: (m[1, w], 0, 0)),
            pl.BlockSpec((1, HIDDEN, D_MODEL), lambda w, m: (m[1, w], 0, 0)),
            pl.BlockSpec((1, 1, D_MODEL), lambda w, m: (m[1, w], 0, 0)),
        ],
        out_specs=pl.BlockSpec((BT, D_MODEL), lambda w, m: (m[0, w], 0)),
    )
    return pl.pallas_call(
        _ffn_body,
        grid_spec=grid_spec,
        out_shape=jax.ShapeDtypeStruct((N_TOK, D_MODEL), jnp.float32),
    )(meta, xs,
      ln_g.reshape(N_EXP, 1, D_MODEL), ln_b.reshape(N_EXP, 1, D_MODEL),
      w1, b1.reshape(N_EXP, 1, HIDDEN), w2, b2.reshape(N_EXP, 1, D_MODEL))


def kernel(x, gate_w, ln_g, ln_b, w1, b1, w2, b2):
    Bx, Tx, D = x.shape
    xt = x.reshape(Bx * Tx, D)
    top1_2d, rank_2d, counts_2d = _router(xt, gate_w)
    top1 = top1_2d[:, 0]
    rank = rank_2d[:, 0]
    counts = counts_2d[0]
    starts = jnp.cumsum(counts) - counts
    er = jnp.arange(N_EXP, dtype=jnp.int32)[None, :]
    start_tok = jnp.sum(jnp.where(top1[:, None] == er, starts[None, :], 0),
                        axis=1)
    inv_perm = (start_tok + rank).astype(jnp.int32)
    idx3 = inv_perm.reshape(SC_WORKERS, N_CHUNK, SC_CHUNK)
    meta = _work_plan(counts)
    xs = _sc_dispatch(xt, idx3)
    ys = _grouped_ffn(meta, xs, ln_g, ln_b, w1, b1, w2, b2)
    y = _sc_combine(ys, idx3)
    return y.reshape(Bx, Tx, D)


# speedup vs baseline: 1.0789x; 1.0789x over previous
import functools

import jax
import jax.numpy as jnp
from jax import lax
from jax.experimental import pallas as pl
from jax.experimental.pallas import tpu as pltpu
from jax.experimental.pallas import tpu_sc as plsc

D_MODEL = 1024
HIDDEN = 2048
N_EXP = 8
N_TOK = 4096
BT = 512
NB = N_TOK // BT
NW = NB + N_EXP - 1

RB = 2048
NRB = N_TOK // RB

SC_WORKERS = 32
ROWS_W = N_TOK // SC_WORKERS
SC_CHUNK = 64
N_CHUNK = ROWS_W // SC_CHUNK


def _router_body(x_ref, gwt_ref, top1_ref, rank_ref, counts_ref, carry):
    b = pl.program_id(0)

    @pl.when(b == 0)
    def _():
        carry[...] = jnp.zeros_like(carry)

    logits = jnp.dot(x_ref[...], gwt_ref[...],
                     preferred_element_type=jnp.float32)
    m = jnp.max(logits, axis=1, keepdims=True)
    col = lax.broadcasted_iota(jnp.int32, (RB, N_EXP), 1)
    top1 = jnp.min(jnp.where(logits == m, col, N_EXP), axis=1)
    oh = (col == top1[:, None]).astype(jnp.int32)
    c = oh
    s = 1
    while s < RB:
        z = jnp.zeros((s, N_EXP), jnp.int32)
        c = c + jnp.concatenate([z, c[:-s]], axis=0)
        s *= 2
    cum = c + carry[...]
    rank = jnp.sum(jnp.where(oh == 1, cum, 0), axis=1) - 1
    top1_ref[...] = top1[:, None]
    rank_ref[...] = rank[:, None]
    carry[...] = cum[-1:, :]
    counts_ref[...] = cum[-1:, :]


def _router(xt, gate_w):
    return pl.pallas_call(
        _router_body,
        grid=(NRB,),
        in_specs=[
            pl.BlockSpec((RB, D_MODEL), lambda i: (i, 0)),
            pl.BlockSpec((D_MODEL, N_EXP), lambda i: (0, 0)),
        ],
        out_specs=[
            pl.BlockSpec((RB, 1), lambda i: (i, 0)),
            pl.BlockSpec((RB, 1), lambda i: (i, 0)),
            pl.BlockSpec((1, N_EXP), lambda i: (0, 0)),
        ],
        out_shape=[
            jax.ShapeDtypeStruct((N_TOK, 1), jnp.int32),
            jax.ShapeDtypeStruct((N_TOK, 1), jnp.int32),
            jax.ShapeDtypeStruct((1, N_EXP), jnp.int32),
        ],
        scratch_shapes=[pltpu.VMEM((1, N_EXP), jnp.int32)],
    )(xt, gate_w.T)


def _work_plan(counts):
    ends = jnp.cumsum(counts)
    starts = ends - counts
    bpos = jnp.arange(NB, dtype=jnp.int32) * BT
    e_lo = jnp.sum((ends[None, :] <= bpos[:, None]).astype(jnp.int32), axis=1)
    e_hi = jnp.sum((ends[None, :] <= (bpos + BT - 1)[:, None]).astype(jnp.int32),
                   axis=1)
    per_block = e_hi - e_lo + 1
    cpb = jnp.cumsum(per_block)
    cpb_prev = cpb - per_block
    nval = cpb[NB - 1]
    w = jnp.arange(NW, dtype=jnp.int32)
    wc = jnp.minimum(w, nval - 1)
    bid = jnp.sum((cpb[None, :] <= wc[:, None]).astype(jnp.int32), axis=1)
    eid = e_lo[bid] + (wc - cpb_prev[bid])
    lo = jnp.maximum(starts[eid], bid * BT) - bid * BT
    hi = jnp.minimum(ends[eid], bid * BT + BT) - bid * BT
    pad = w >= nval
    lo = jnp.where(pad, 0, lo).astype(jnp.int32)
    hi = jnp.where(pad, 0, hi).astype(jnp.int32)
    first = jnp.concatenate([
        jnp.ones((1,), jnp.int32),
        (bid[1:] != bid[:-1]).astype(jnp.int32),
    ])
    return jnp.stack([bid.astype(jnp.int32), eid.astype(jnp.int32),
                      lo, hi, first])


def _sc_mesh():
    return plsc.VectorSubcoreMesh(core_axis_name="c", subcore_axis_name="s")


def _dispatch_body(x_hbm, idx_hbm, out_hbm, idx_v, rows_v, sem):
    wid = lax.axis_index("s") * 2 + lax.axis_index("c")
    base = wid * ROWS_W
    pltpu.sync_copy(idx_hbm.at[wid], idx_v)
    for ch in range(N_CHUNK):
        pltpu.sync_copy(x_hbm.at[pl.ds(base + ch * SC_CHUNK, SC_CHUNK)], rows_v)
        pltpu.async_copy(rows_v, out_hbm.at[idx_v.at[ch]], sem).wait()


def _sc_dispatch(xt, idx3):
    k = functools.partial(
        pl.kernel,
        out_type=jax.ShapeDtypeStruct((N_TOK, D_MODEL), jnp.float32),
        mesh=_sc_mesh(),
        scratch_types=[
            pltpu.VMEM((N_CHUNK, SC_CHUNK), jnp.int32),
            pltpu.VMEM((SC_CHUNK, D_MODEL), jnp.float32),
            pltpu.SemaphoreType.DMA,
        ],
    )(_dispatch_body)
    return k(xt, idx3)


def _combine_body(ys_hbm, idx_hbm, out_hbm, idx_v, rows_v, sem):
    wid = lax.axis_index("s") * 2 + lax.axis_index("c")
    base = wid * ROWS_W
    pltpu.sync_copy(idx_hbm.at[wid], idx_v)
    for ch in range(N_CHUNK):
        pltpu.async_copy(ys_hbm.at[idx_v.at[ch]], rows_v, sem).wait()
        pltpu.sync_copy(rows_v, out_hbm.at[pl.ds(base + ch * SC_CHUNK, SC_CHUNK)])


def _sc_combine(ys, idx3):
    k = functools.partial(
        pl.kernel,
        out_type=jax.ShapeDtypeStruct((N_TOK, D_MODEL), jnp.float32),
        mesh=_sc_mesh(),
        scratch_types=[
            pltpu.VMEM((N_CHUNK, SC_CHUNK), jnp.int32),
            pltpu.VMEM((SC_CHUNK, D_MODEL), jnp.float32),
            pltpu.SemaphoreType.DMA,
        ],
    )(_combine_body)
    return k(ys, idx3)


def _ffn_body(meta_ref, xs_ref, lng_ref, lnb_ref, w1_ref, b1_ref, w2_ref,
              b2_ref, out_ref):
    w = pl.program_id(0)
    lo = meta_ref[2, w]
    hi = meta_ref[3, w]
    first = meta_ref[4, w]

    @pl.when(first == 1)
    def _():
        out_ref[...] = jnp.zeros_like(out_ref)

    @pl.when(hi > lo)
    def _():
        xb = xs_ref[...]
        mu = jnp.mean(xb, axis=1, keepdims=True)
        var = jnp.mean(jnp.square(xb - mu), axis=1, keepdims=True)
        xhat = (xb - mu) * lax.rsqrt(var + 1e-5)
        ln = xhat * lng_ref[0, 0] + lnb_ref[0, 0]
        h = jnp.dot(ln, w1_ref[0], preferred_element_type=jnp.float32)
        h = h + b1_ref[0, 0]
        h = 0.5 * h * (1.0 + lax.erf(h * 0.7071067811865476))
        o = jnp.dot(h, w2_ref[0], preferred_element_type=jnp.float32)
        o = o + b2_ref[0, 0]
        ridx = lax.broadcasted_iota(jnp.int32, (BT, 1), 0)
        mask = (ridx >= lo) & (ridx < hi)
        out_ref[...] += jnp.where(mask, o, 0.0)


def _grouped_ffn(meta, xs, ln_g, ln_b, w1, b1, w2, b2):
    grid_spec = pltpu.PrefetchScalarGridSpec(
        num_scalar_prefetch=1,
        grid=(NW,),
        in_specs=[
            pl.BlockSpec((BT, D_MODEL), lambda w, m: (m[0, w], 0)),
            pl.BlockSpec((1, 1, D_MODEL), lambda w, m: (m[1, w], 0, 0)),
            pl.BlockSpec((1, 1, D_MODEL), lambda w, m: (m[1, w], 0, 0)),
            pl.BlockSpec((1, D_MODEL, HIDDEN), lambda w, m: (m[1, w], 0, 0)),
            pl.BlockSpec((1, 1, HIDDEN), lambda w, m: (m[1, w], 0, 0)),
            pl.BlockSpec((1, HIDDEN, D_MODEL), lambda w, m: (m[1, w], 0, 0)),
            pl.BlockSpec((1, 1, D_MODEL), lambda w, m: (m[1, w], 0, 0)),
        ],
        out_specs=pl.BlockSpec((BT, D_MODEL), lambda w, m: (m[0, w], 0)),
    )
    return pl.pallas_call(
        _ffn_body,
        grid_spec=grid_spec,
        out_shape=jax.ShapeDtypeStruct((N_TOK, D_MODEL), jnp.float32),
    )(meta, xs,
      ln_g.reshape(N_EXP, 1, D_MODEL), ln_b.reshape(N_EXP, 1, D_MODEL),
      w1, b1.reshape(N_EXP, 1, HIDDEN), w2, b2.reshape(N_EXP, 1, D_MODEL))


def kernel(x, gate_w, ln_g, ln_b, w1, b1, w2, b2):
    Bx, Tx, D = x.shape
    xt = x.reshape(Bx * Tx, D)
    top1_2d, rank_2d, counts_2d = _router(xt, gate_w)
    top1 = top1_2d[:, 0]
    rank = rank_2d[:, 0]
    counts = counts_2d[0]
    starts = jnp.cumsum(counts) - counts
    er = jnp.arange(N_EXP, dtype=jnp.int32)[None, :]
    start_tok = jnp.sum(jnp.where(top1[:, None] == er, starts[None, :], 0),
                        axis=1)
    inv_perm = (start_tok + rank).astype(jnp.int32)
    idx3 = inv_perm.reshape(SC_WORKERS, N_CHUNK, SC_CHUNK)
    meta = _work_plan(counts)
    xs = _sc_dispatch(xt, idx3)
    ys = _grouped_ffn(meta, xs, ln_g, ln_b, w1, b1, w2, b2)
    y = _sc_combine(ys, idx3)
    return y.reshape(Bx, Tx, D)


# speedup vs baseline: 1.0866x; 1.0072x over previous
import functools

import jax
import jax.numpy as jnp
from jax import lax
from jax.experimental import pallas as pl
from jax.experimental.pallas import tpu as pltpu
from jax.experimental.pallas import tpu_sc as plsc

D_MODEL = 1024
HIDDEN = 2048
N_EXP = 8
N_TOK = 4096
BT = 512
NB = N_TOK // BT
NW = NB + N_EXP - 1

RB = 2048
NRB = N_TOK // RB

SC_WORKERS = 32
ROWS_W = N_TOK // SC_WORKERS
SC_CHUNK = 64
N_CHUNK = ROWS_W // SC_CHUNK


def _lane_cumsum(v, n):
    s = 1
    while s < n:
        z = jnp.zeros(v.shape[:-1] + (s,), v.dtype)
        v = v + jnp.concatenate([z, v[..., :-s]], axis=-1)
        s *= 2
    return v


def _router_body(x_ref, gwt_ref, inv_ref, meta_ref):
    logits = jnp.dot(x_ref[...], gwt_ref[...],
                     preferred_element_type=jnp.float32)
    m = jnp.max(logits, axis=1, keepdims=True)
    col = lax.broadcasted_iota(jnp.int32, (N_TOK, N_EXP), 1)
    top1 = jnp.min(jnp.where(logits == m, col, N_EXP), axis=1,
                   keepdims=True)
    oh = (col == top1).astype(jnp.int32)
    c = oh
    s = 1
    while s < N_TOK:
        z = jnp.zeros((s, N_EXP), jnp.int32)
        c = c + jnp.concatenate([z, c[:-s]], axis=0)
        s *= 2
    counts = c[N_TOK - 1:, :]
    ends = _lane_cumsum(counts, N_EXP)
    starts = ends - counts
    rank = jnp.sum(jnp.where(oh == 1, c, 0), axis=1, keepdims=True) - 1
    start_tok = jnp.sum(oh * starts, axis=1, keepdims=True)
    inv_ref[...] = start_tok + rank

    ident = (lax.broadcasted_iota(jnp.int32, (N_EXP, N_EXP), 0) ==
             lax.broadcasted_iota(jnp.int32, (N_EXP, N_EXP), 1)).astype(jnp.int32)
    ends_col = jnp.sum(ident * ends, axis=1, keepdims=True)
    bpos = lax.broadcasted_iota(jnp.int32, (1, NB), 1) * BT
    e_lo = jnp.sum((ends_col <= bpos).astype(jnp.int32), axis=0,
                   keepdims=True)
    e_hi = jnp.sum((ends_col <= bpos + (BT - 1)).astype(jnp.int32), axis=0,
                   keepdims=True)
    per_block = e_hi - e_lo + 1
    cpb = _lane_cumsum(per_block, NB)
    cpb_prev = cpb - per_block
    nval = cpb[:, NB - 1:]
    w = lax.broadcasted_iota(jnp.int32, (NW, 1), 0)
    wc = jnp.minimum(w, nval - 1)
    bid = jnp.sum((cpb <= wc).astype(jnp.int32), axis=1, keepdims=True)
    ohb = (lax.broadcasted_iota(jnp.int32, (NW, NB), 1) == bid).astype(jnp.int32)
    e_lo_w = jnp.sum(ohb * e_lo, axis=1, keepdims=True)
    cpb_prev_w = jnp.sum(ohb * cpb_prev, axis=1, keepdims=True)
    eid = e_lo_w + wc - cpb_prev_w
    ohe = (lax.broadcasted_iota(jnp.int32, (NW, N_EXP), 1) == eid).astype(jnp.int32)
    starts_w = jnp.sum(ohe * starts, axis=1, keepdims=True)
    ends_w = jnp.sum(ohe * ends, axis=1, keepdims=True)
    lo = jnp.maximum(starts_w, bid * BT) - bid * BT
    hi = jnp.minimum(ends_w, bid * BT + BT) - bid * BT
    pad = w >= nval
    lo = jnp.where(pad, 0, lo)
    hi = jnp.where(pad, 0, hi)
    prev_bid = jnp.concatenate([jnp.full((1, 1), -1, jnp.int32), bid[:-1]],
                               axis=0)
    first = (bid != prev_bid).astype(jnp.int32)
    zpad = jnp.zeros((NW, 3), jnp.int32)
    meta_ref[...] = jnp.concatenate([bid, eid, lo, hi, first, zpad], axis=1)


def _router(xt, gate_w):
    return pl.pallas_call(
        _router_body,
        grid=(1,),
        in_specs=[
            pl.BlockSpec((N_TOK, D_MODEL), lambda i: (0, 0)),
            pl.BlockSpec((D_MODEL, N_EXP), lambda i: (0, 0)),
        ],
        out_specs=[
            pl.BlockSpec((N_TOK, 1), lambda i: (0, 0)),
            pl.BlockSpec((NW, 8), lambda i: (0, 0)),
        ],
        out_shape=[
            jax.ShapeDtypeStruct((N_TOK, 1), jnp.int32),
            jax.ShapeDtypeStruct((NW, 8), jnp.int32),
        ],
    )(xt, gate_w.T)


def _sc_mesh():
    return plsc.VectorSubcoreMesh(core_axis_name="c", subcore_axis_name="s")


def _dispatch_body(x_hbm, idx_hbm, out_hbm, idx_v, rows_v, sem):
    wid = lax.axis_index("s") * 2 + lax.axis_index("c")
    base = wid * ROWS_W
    pltpu.sync_copy(idx_hbm.at[wid], idx_v)
    for ch in range(N_CHUNK):
        pltpu.sync_copy(x_hbm.at[pl.ds(base + ch * SC_CHUNK, SC_CHUNK)], rows_v)
        pltpu.async_copy(rows_v, out_hbm.at[idx_v.at[ch]], sem).wait()


def _sc_dispatch(xt, idx3):
    k = functools.partial(
        pl.kernel,
        out_type=jax.ShapeDtypeStruct((N_TOK, D_MODEL), jnp.float32),
        mesh=_sc_mesh(),
        scratch_types=[
            pltpu.VMEM((N_CHUNK, SC_CHUNK), jnp.int32),
            pltpu.VMEM((SC_CHUNK, D_MODEL), jnp.float32),
            pltpu.SemaphoreType.DMA,
        ],
    )(_dispatch_body)
    return k(xt, idx3)


def _combine_body(ys_hbm, idx_hbm, out_hbm, idx_v, rows_v, sem):
    wid = lax.axis_index("s") * 2 + lax.axis_index("c")
    base = wid * ROWS_W
    pltpu.sync_copy(idx_hbm.at[wid], idx_v)
    for ch in range(N_CHUNK):
        pltpu.async_copy(ys_hbm.at[idx_v.at[ch]], rows_v, sem).wait()
        pltpu.sync_copy(rows_v, out_hbm.at[pl.ds(base + ch * SC_CHUNK, SC_CHUNK)])


def _sc_combine(ys, idx3):
    k = functools.partial(
        pl.kernel,
        out_type=jax.ShapeDtypeStruct((N_TOK, D_MODEL), jnp.float32),
        mesh=_sc_mesh(),
        scratch_types=[
            pltpu.VMEM((N_CHUNK, SC_CHUNK), jnp.int32),
            pltpu.VMEM((SC_CHUNK, D_MODEL), jnp.float32),
            pltpu.SemaphoreType.DMA,
        ],
    )(_combine_body)
    return k(ys, idx3)


def _ffn_body(meta_ref, xs_ref, lng_ref, lnb_ref, w1_ref, b1_ref, w2_ref,
              b2_ref, out_ref):
    w = pl.program_id(0)
    lo = meta_ref[w, 2]
    hi = meta_ref[w, 3]
    first = meta_ref[w, 4]

    @pl.when(first == 1)
    def _():
        out_ref[...] = jnp.zeros_like(out_ref)

    @pl.when(hi > lo)
    def _():
        xb = xs_ref[...]
        mu = jnp.mean(xb, axis=1, keepdims=True)
        var = jnp.mean(jnp.square(xb - mu), axis=1, keepdims=True)
        xhat = (xb - mu) * lax.rsqrt(var + 1e-5)
        ln = xhat * lng_ref[0, 0] + lnb_ref[0, 0]
        h = jnp.dot(ln, w1_ref[0], preferred_element_type=jnp.float32)
        h = h + b1_ref[0, 0]
        h = 0.5 * h * (1.0 + lax.erf(h * 0.7071067811865476))
        o = jnp.dot(h, w2_ref[0], preferred_element_type=jnp.float32)
        o = o + b2_ref[0, 0]
        ridx = lax.broadcasted_iota(jnp.int32, (BT, 1), 0)
        mask = (ridx >= lo) & (ridx < hi)
        out_ref[...] += jnp.where(mask, o, 0.0)


def _grouped_ffn(meta, xs, ln_g, ln_b, w1, b1, w2, b2):
    grid_spec = pltpu.PrefetchScalarGridSpec(
        num_scalar_prefetch=1,
        grid=(NW,),
        in_specs=[
            pl.BlockSpec((BT, D_MODEL), lambda w, m: (m[w, 0], 0)),
            pl.BlockSpec((1, 1, D_MODEL), lambda w, m: (m[w, 1], 0, 0)),
            pl.BlockSpec((1, 1, D_MODEL), lambda w, m: (m[w, 1], 0, 0)),
            pl.BlockSpec((1, D_MODEL, HIDDEN), lambda w, m: (m[w, 1], 0, 0)),
            pl.BlockSpec((1, 1, HIDDEN), lambda w, m: (m[w, 1], 0, 0)),
            pl.BlockSpec((1, HIDDEN, D_MODEL), lambda w, m: (m[w, 1], 0, 0)),
            pl.BlockSpec((1, 1, D_MODEL), lambda w, m: (m[w, 1], 0, 0)),
        ],
        out_specs=pl.BlockSpec((BT, D_MODEL), lambda w, m: (m[w, 0], 0)),
    )
    return pl.pallas_call(
        _ffn_body,
        grid_spec=grid_spec,
        out_shape=jax.ShapeDtypeStruct((N_TOK, D_MODEL), jnp.float32),
    )(meta, xs,
      ln_g.reshape(N_EXP, 1, D_MODEL), ln_b.reshape(N_EXP, 1, D_MODEL),
      w1, b1.reshape(N_EXP, 1, HIDDEN), w2, b2.reshape(N_EXP, 1, D_MODEL))


def kernel(x, gate_w, ln_g, ln_b, w1, b1, w2, b2):
    Bx, Tx, D = x.shape
    xt = x.reshape(Bx * Tx, D)
    inv_2d, meta = _router(xt, gate_w)
    idx3 = inv_2d.reshape(SC_WORKERS, N_CHUNK, SC_CHUNK)
    xs = _sc_dispatch(xt, idx3)
    ys = _grouped_ffn(meta, xs, ln_g, ln_b, w1, b1, w2, b2)
    y = _sc_combine(ys, idx3)
    return y.reshape(Bx, Tx, D)
